# baseline (device time: 115973 ns/iter reference)
import jax
import jax.numpy as jnp
import numpy as np
from jax import lax
from jax.experimental import pallas as pl
from jax.experimental.pallas import tpu as pltpu

N_DEV = 4
B = 2
S_LOC = 512
D = 1024
HQ = 8
DH = 128
HALF = D // 2
SCALE = 0.08838834764831843

_PERM = np.concatenate(
    [
        h * DH + np.concatenate([np.arange(0, DH, 2), np.arange(1, DH, 2)])
        for h in range(HQ)
    ]
)

_CompilerParams = getattr(pltpu, "CompilerParams", None) or getattr(
    pltpu, "TPUCompilerParams"
)


def kernel(x, Wq, Wk, Wv, Wo):
    my = lax.axis_index("i")

    pos = (my * S_LOC + jnp.arange(S_LOC))[:, None].astype(jnp.float32)
    inv = jnp.asarray(
        1.0 / (10000.0 ** (np.arange(0, DH, 2) / DH)), dtype=jnp.float32
    )
    ang = pos * inv[None, :]
    cos = jnp.concatenate([jnp.cos(ang), jnp.cos(ang)], axis=-1).astype(
        jnp.bfloat16
    )
    sin = jnp.concatenate([jnp.sin(ang), jnp.sin(ang)], axis=-1).astype(
        jnp.bfloat16
    )

    Wq_p = Wq[:, _PERM]
    Wk_p = Wk[:, _PERM]

    def body(
        x_ref,
        wq_ref,
        wk_ref,
        wv_ref,
        wo_ref,
        cos_ref,
        sin_ref,
        out_ref,
        rbuf,
        lbuf,
        acc_ref,
        r_send,
        r_recv,
        l_send,
        l_recv,
    ):
        my_pos = lax.axis_index("i")
        left = (my_pos - 1) % N_DEV
        right = (my_pos + 1) % N_DEV

        barrier_sem = pltpu.get_barrier_semaphore()
        for nbr in (left, right):
            pl.semaphore_signal(
                barrier_sem,
                inc=1,
                device_id=(nbr,),
                device_id_type=pl.DeviceIdType.MESH,
            )
        pl.semaphore_wait(barrier_sem, 2)

        cos_b = cos_ref[...]
        sin_b = sin_ref[...]
        wk_bf = wk_ref[...].astype(jnp.bfloat16)
        wv_bf = wv_ref[...].astype(jnp.bfloat16)

        def rope_batch(t):
            blocks = []
            for hd in range(HQ):
                tb = t[:, hd * DH : (hd + 1) * DH]
                tb_rot = jnp.concatenate(
                    [-tb[:, DH // 2 :], tb[:, : DH // 2]], axis=-1
                )
                blocks.append(tb * cos_b + tb_rot * sin_b)
            return jnp.concatenate(blocks, axis=-1)

        xcache = {}

        def proj_batch(b, w_bf, rope):
            if b not in xcache:
                xcache[b] = x_ref[b, :, :].astype(jnp.bfloat16)
            t = jnp.dot(xcache[b], w_bf, preferred_element_type=jnp.float32)
            t = t.astype(jnp.bfloat16)
            return rope_batch(t) if rope else t

        started = []

        def rows(s):
            return pl.ds(s * S_LOC, S_LOC)

        def start_hop(h, s):
            r = pltpu.make_async_remote_copy(
                src_ref=rbuf.at[h, rows(s)],
                dst_ref=rbuf.at[h + 1, rows(s)],
                send_sem=r_send.at[h, s],
                recv_sem=r_recv.at[h, s],
                device_id=(right,),
                device_id_type=pl.DeviceIdType.MESH,
            )
            l = pltpu.make_async_remote_copy(
                src_ref=lbuf.at[h, rows(s)],
                dst_ref=lbuf.at[h + 1, rows(s)],
                send_sem=l_send.at[h, s],
                recv_sem=l_recv.at[h, s],
                device_id=(left,),
                device_id_type=pl.DeviceIdType.MESH,
            )
            r.start()
            l.start()
            started.append(r)
            started.append(l)
            return r, l

        def wait_arrival(pair):
            pair[0].wait_recv()
            pair[1].wait_recv()

        hops = {0: [None, None]}
        for s in range(B):
            kb = proj_batch(s, wk_bf, rope=True)
            vb = proj_batch(s, wv_bf, rope=False)
            rbuf[0, rows(s), :HALF] = kb[:, :HALF]
            rbuf[0, rows(s), HALF:] = vb[:, :HALF]
            lbuf[0, rows(s), :HALF] = kb[:, HALF:]
            lbuf[0, rows(s), HALF:] = vb[:, HALF:]
            hops[0][s] = start_hop(0, s)

        qscale = jnp.bfloat16(SCALE * 1.4426950408889634)
        wq_bf = wq_ref[...].astype(jnp.bfloat16)
        q_b = [(proj_batch(b, wq_bf, rope=True) * qscale) for b in range(B)]

        l_st = [[None] * HQ for _ in range(B)]

        def fold(b, hd, kbh, vbh, first):
            rs = slice(b * S_LOC, (b + 1) * S_LOC)
            cs = slice(hd * DH, (hd + 1) * DH)
            qbh = q_b[b][:, cs]
            s = lax.dot_general(
                qbh,
                kbh,
                (((1,), (1,)), ((), ())),
                preferred_element_type=jnp.float32,
            )
            p = jnp.exp2(s)
            pv = jnp.dot(
                p.astype(jnp.bfloat16), vbh, preferred_element_type=jnp.float32
            )
            lsum = jnp.sum(p, axis=-1, keepdims=True)
            if first:
                l_st[b][hd] = lsum
                acc_ref[rs, cs] = pv
            else:
                l_st[b][hd] = l_st[b][hd] + lsum
                acc_ref[rs, cs] = acc_ref[rs, cs] + pv

        def fold_sub(slot, b, first=False):
            rsl = rbuf[slot, b * S_LOC : (b + 1) * S_LOC, :]
            lsl = lbuf[slot, b * S_LOC : (b + 1) * S_LOC, :]
            for j in range(HQ // 2):
                cs = slice(j * DH, (j + 1) * DH)
                vs = slice(HALF + j * DH, HALF + (j + 1) * DH)
                fold(b, j, rsl[:, cs], rsl[:, vs], first)
                fold(b, 4 + j, lsl[:, cs], lsl[:, vs], first)

        def finish_batch(b, wo_bf):
            rs = pl.ds(b * S_LOC, S_LOC)
            for hd in range(HQ):
                cs = slice(hd * DH, (hd + 1) * DH)
                ctx = acc_ref[b * S_LOC : (b + 1) * S_LOC, cs] / l_st[b][hd]
                rbuf[0, rs, cs] = ctx.astype(jnp.bfloat16)
            out_b = jnp.dot(
                rbuf[0, rs, :], wo_bf, preferred_element_type=jnp.float32
            )
            out_ref[b, :, :] = out_b

        fold_sub(0, 0, first=True)
        wait_arrival(hops[0][0])
        hops[1] = [start_hop(1, 0), None]
        fold_sub(0, 1, first=True)
        wait_arrival(hops[0][1])
        hops[1][1] = start_hop(1, 1)
        fold_sub(1, 0)
        wait_arrival(hops[1][0])
        hops[2] = [start_hop(2, 0), None]
        fold_sub(1, 1)
        wait_arrival(hops[1][1])
        hops[2][1] = start_hop(2, 1)
        fold_sub(2, 0)
        wo_bf = wo_ref[...].astype(jnp.bfloat16)
        wait_arrival(hops[2][0])
        fold_sub(2, 1)
        fold_sub(3, 0)
        finish_batch(0, wo_bf)
        wait_arrival(hops[2][1])
        fold_sub(3, 1)
        finish_batch(1, wo_bf)

        for rdma in started:
            rdma.wait_send()

    return pl.pallas_call(
        body,
        out_shape=jax.ShapeDtypeStruct((B, S_LOC, D), jnp.float32),
        in_specs=[pl.BlockSpec(memory_space=pltpu.VMEM)] * 7,
        out_specs=pl.BlockSpec(memory_space=pltpu.VMEM),
        scratch_shapes=[
            pltpu.VMEM((N_DEV, B * S_LOC, D), jnp.bfloat16),
            pltpu.VMEM((N_DEV, B * S_LOC, D), jnp.bfloat16),
            pltpu.VMEM((B * S_LOC, D), jnp.float32),
            pltpu.SemaphoreType.DMA((N_DEV - 1, B)),
            pltpu.SemaphoreType.DMA((N_DEV - 1, B)),
            pltpu.SemaphoreType.DMA((N_DEV - 1, B)),
            pltpu.SemaphoreType.DMA((N_DEV - 1, B)),
        ],
        compiler_params=_CompilerParams(
            collective_id=0, vmem_limit_bytes=100 * 1024 * 1024
        ),
    )(x, Wq_p, Wk_p, Wv, Wo, cos, sin)
